# baseline (speedup 1.0000x reference)
import functools

import jax
import jax.numpy as jnp
import numpy as np
from jax import lax
from jax.experimental import pallas as pl
from jax.experimental.pallas import tpu as pltpu

IMG = 512
STRIDE = 16
FH = FW = IMG // STRIDE
SIZES = (32.0, 64.0, 128.0, 256.0)
RATIOS = (0.5, 1.0, 2.0)
A = 12
PRE_NMS = 1000
POST_NMS = 256
NMS_TH = 0.7
POOL = 7
NUM_CLASSES = 91
BBOX_CLAMP = float(np.log(1000.0 / 16.0))


def _fiota(shape, dim):
    return jax.lax.broadcasted_iota(jnp.int32, shape, dim).astype(jnp.float32)
NLOC = FH * FW
NANCH = NLOC * A
KPAD = 1024


def _make_anchors_np():
    scales = np.array(SIZES, np.float32)
    rat = np.array(RATIOS, np.float32)
    h_r = np.sqrt(rat)
    w_r = 1.0 / h_r
    ws = (w_r[:, None] * scales[None, :]).reshape(-1)
    hs = (h_r[:, None] * scales[None, :]).reshape(-1)
    base = np.stack([-ws, -hs, ws, hs], axis=1) / 2.0
    sy, sx = np.meshgrid(np.arange(FH) * STRIDE, np.arange(FW) * STRIDE, indexing="ij")
    shifts = np.stack([sx, sy, sx, sy], axis=-1).reshape(-1, 4).astype(np.float32)
    return (shifts[:, None, :] + base[None, :, :]).reshape(-1, 4)


_ANCHORS = _make_anchors_np()


def _mm_body(x_ref, w_ref, b_ref, o_ref, *, ksteps, act):
    @pl.when(pl.program_id(2) == 0)
    def _init():
        o_ref[...] = jnp.zeros_like(o_ref)

    o_ref[...] += jnp.dot(x_ref[...].astype(jnp.bfloat16),
                          w_ref[...].astype(jnp.bfloat16),
                          preferred_element_type=jnp.float32)

    @pl.when(pl.program_id(2) == ksteps - 1)
    def _fin():
        r = o_ref[...] + b_ref[...]
        if act:
            r = jnp.maximum(r, 0.0)
        o_ref[...] = r


def _matmul(x, w, bias, act, bm, bn, bk):
    M, K = x.shape
    N = w.shape[1]
    grid = (M // bm, N // bn, K // bk)
    return pl.pallas_call(
        functools.partial(_mm_body, ksteps=grid[2], act=act),
        grid=grid,
        in_specs=[
            pl.BlockSpec((bm, bk), lambda i, j, k: (i, k)),
            pl.BlockSpec((bk, bn), lambda i, j, k: (k, j)),
            pl.BlockSpec((1, bn), lambda i, j, k: (0, j)),
        ],
        out_specs=pl.BlockSpec((bm, bn), lambda i, j, k: (i, j)),
        out_shape=jax.ShapeDtypeStruct((M, N), jnp.float32),
        compiler_params=pltpu.CompilerParams(
            dimension_semantics=("parallel", "parallel", "arbitrary")),
    )(x, w, bias.reshape(1, N))


def _im2col(x_hwc, ksz, stride, pad):
    H, W, C = x_hwc.shape
    Ho = (H + 2 * pad - ksz) // stride + 1
    Wo = (W + 2 * pad - ksz) // stride + 1
    xp = jnp.pad(x_hwc, ((pad, pad), (pad, pad), (0, 0)))
    cols = []
    for dy in range(ksz):
        for dx in range(ksz):
            sl = xp[dy: dy + stride * (Ho - 1) + 1: stride,
                    dx: dx + stride * (Wo - 1) + 1: stride, :]
            cols.append(sl.reshape(Ho * Wo, C))
    return jnp.concatenate(cols, axis=1)


def _wconv(w):
    O, I, kh, kw = w.shape
    return jnp.transpose(w, (2, 3, 1, 0)).reshape(kh * kw * I, O)


def _maxpool_body(x_ref, o_ref):
    o_ref[...] = jnp.max(x_ref[...], axis=0)


def _maxpool(y, H, W, C):
    x = y.reshape(H, W, C)
    xp = jnp.pad(x, ((1, 1), (1, 1), (0, 0)), constant_values=-jnp.inf)
    Ho = H // 2
    views = []
    for dy in range(3):
        for dx in range(3):
            sl = xp[dy: dy + 2 * (Ho - 1) + 1: 2,
                    dx: dx + 2 * (Ho - 1) + 1: 2, :]
            views.append(sl.reshape(Ho * Ho, C))
    stacked = jnp.stack(views, axis=0)
    M = Ho * Ho
    bm = 2048
    return pl.pallas_call(
        _maxpool_body,
        grid=(M // bm,),
        in_specs=[pl.BlockSpec((9, bm, C), lambda i: (0, i, 0))],
        out_specs=pl.BlockSpec((bm, C), lambda i: (i, 0)),
        out_shape=jax.ShapeDtypeStruct((M, C), jnp.float32),
        compiler_params=pltpu.CompilerParams(
            dimension_semantics=("parallel",)),
    )(stacked)


def _rank_body(scol_ref, srow_ref, o_ref):
    pid = pl.program_id(0)
    sc = scol_ref[...]
    i_idx = (_fiota((KPAD, 1), 0)
             + pid.astype(jnp.float32) * KPAD)
    acc = jnp.zeros((KPAD, 1), jnp.float32)
    for jb in range(NANCH // KPAD):
        sr = srow_ref[0:1, jb * KPAD:(jb + 1) * KPAD]
        j_idx = (_fiota((1, KPAD), 1)
                 + float(jb * KPAD))
        gt = (sr > sc).astype(jnp.float32)
        tie = ((sr == sc) & (j_idx < i_idx)).astype(jnp.float32)
        acc += jnp.sum(gt + tie, axis=1, keepdims=True)
    o_ref[...] = acc


def _rank(scores):
    scol = scores.reshape(NANCH, 1)
    srow = scores.reshape(1, NANCH)
    return pl.pallas_call(
        _rank_body,
        grid=(NANCH // KPAD,),
        in_specs=[
            pl.BlockSpec((KPAD, 1), lambda i: (i, 0)),
            pl.BlockSpec((1, NANCH), lambda i: (0, 0)),
        ],
        out_specs=pl.BlockSpec((KPAD, 1), lambda i: (i, 0)),
        out_shape=jax.ShapeDtypeStruct((NANCH, 1), jnp.float32),
        compiler_params=pltpu.CompilerParams(
            dimension_semantics=("parallel",)),
    )(scol, srow)


def _decode_parts(d0, d1, d2, d3, a0, a1, a2, a3):
    wa = a2 - a0
    ha = a3 - a1
    cxa = a0 + 0.5 * wa
    cya = a1 + 0.5 * ha
    dw = jnp.minimum(d2, BBOX_CLAMP)
    dh = jnp.minimum(d3, BBOX_CLAMP)
    cx = d0 * wa + cxa
    cy = d1 * ha + cya
    w = wa * jnp.exp(dw)
    h = ha * jnp.exp(dh)
    clip = lambda v: jnp.clip(v, 0.0, float(IMG))
    return (clip(cx - 0.5 * w), clip(cy - 0.5 * h),
            clip(cx + 0.5 * w), clip(cy + 0.5 * h))


def _nms_body(rkc_ref, rkr_ref, dcat_ref, dcatT_ref, props_ref, iou_ref):
    kcol = _fiota((KPAD, 1), 0)
    krow = _fiota((1, KPAD), 1)

    def gbody(jb, carry):
        cand, candT = carry
        off = jb * KPAD
        rkb_r = rkr_ref[0:1, pl.ds(off, KPAD)]
        rkb_c = rkc_ref[pl.ds(off, KPAD), :]
        oh = (rkb_r == kcol).astype(jnp.float32)
        ohT = (rkb_c == krow).astype(jnp.float32)
        cand += jnp.dot(oh, dcat_ref[pl.ds(off, KPAD), :],
                        preferred_element_type=jnp.float32,
                        precision=jax.lax.Precision.HIGHEST)
        candT += jnp.dot(dcatT_ref[:, pl.ds(off, KPAD)], ohT,
                         preferred_element_type=jnp.float32,
                         precision=jax.lax.Precision.HIGHEST)
        return cand, candT

    cand, candT = lax.fori_loop(
        0, NANCH // KPAD, gbody,
        (jnp.zeros((KPAD, 8), jnp.float32), jnp.zeros((8, KPAD), jnp.float32)))

    cc = [cand[:, c:c + 1] for c in range(8)]
    x0c, y0c, x1c, y1c = _decode_parts(*cc)
    cr = [candT[c:c + 1, :] for c in range(8)]
    x0r, y0r, x1r, y1r = _decode_parts(*cr)
    BR = 128
    for rb in range(KPAD // BR):
        sl = slice(rb * BR, (rb + 1) * BR)
        xc0, yc0, xc1, yc1 = x0c[sl], y0c[sl], x1c[sl], y1c[sl]
        area_c = (xc1 - xc0) * (yc1 - yc0)
        area_r = (x1r - x0r) * (y1r - y0r)
        iw = jnp.maximum(jnp.minimum(xc1, x1r) - jnp.maximum(xc0, x0r), 0.0)
        ih = jnp.maximum(jnp.minimum(yc1, y1r) - jnp.maximum(yc0, y0r), 0.0)
        inter = iw * ih
        iou_ref[sl, :] = inter / (area_c + area_r - inter + 1e-9)

    lane = krow
    keep0 = (lane < float(PRE_NMS)).astype(jnp.float32)

    def body(i, keep):
        row = iou_ref[pl.ds(i, 1), :]
        fi = i.astype(jnp.float32)
        keep_i = jnp.sum(jnp.where(lane == fi, keep, 0.0))
        mask = keep_i * (row > NMS_TH).astype(jnp.float32) \
            * (lane > fi).astype(jnp.float32)
        return keep * (1.0 - mask)

    keep = lax.fori_loop(0, PRE_NMS, body, keep0)
    pk = keep
    d = 1
    while d < KPAD:
        pk = pk + jnp.concatenate(
            [jnp.zeros((1, d), jnp.float32), pk[:, :KPAD - d]], axis=1)
        d *= 2
    pk = pk - keep
    ktot = jnp.sum(keep)
    pos_row = jnp.where(keep > 0.5, pk, ktot + (lane - pk))
    boxes4 = jnp.concatenate([x0c, y0c, x1c, y1c], axis=1)
    k256 = _fiota((POST_NMS, 1), 0)
    ohp = (pos_row == k256).astype(jnp.float32)
    props_ref[...] = jnp.dot(ohp, boxes4,
                             preferred_element_type=jnp.float32,
                             precision=jax.lax.Precision.HIGHEST)


def _nms_select(ranks_col, dcat):
    ranks_row = ranks_col.reshape(1, NANCH)
    dcatT = dcat.T
    return pl.pallas_call(
        _nms_body,
        in_specs=[
            pl.BlockSpec((NANCH, 1), lambda: (0, 0)),
            pl.BlockSpec((1, NANCH), lambda: (0, 0)),
            pl.BlockSpec((NANCH, 8), lambda: (0, 0)),
            pl.BlockSpec((8, NANCH), lambda: (0, 0)),
        ],
        out_specs=pl.BlockSpec((POST_NMS, 4), lambda: (0, 0)),
        out_shape=jax.ShapeDtypeStruct((POST_NMS, 4), jnp.float32),
        scratch_shapes=[pltpu.VMEM((KPAD, KPAD), jnp.float32)],
    )(ranks_col, ranks_row, dcat, dcatT)


def _roi_body(bxr_ref, feat_ref, o_ref):
    BM = o_ref.shape[0]
    SB = 392
    for sb in range(BM // SB):
        sl = slice(sb * SB, (sb + 1) * SB)
        _roi_sub(bxr_ref, feat_ref, o_ref, sl, SB)


def _roi_sub(bxr_ref, feat_ref, o_ref, sl, BM):
    it = jax.lax.broadcasted_iota(jnp.int32, (BM, 1), 0)
    p = it % 49
    py = (p // 7).astype(jnp.float32)
    px = (p % 7).astype(jnp.float32)
    b = [bxr_ref[sl, c:c + 1] * (1.0 / STRIDE) for c in range(4)]
    gy = (py + 0.5) / POOL
    gx = (px + 0.5) / POOL
    yy = b[1] + gy * (b[3] - b[1]) - 0.5
    xx = b[0] + gx * (b[2] - b[0]) - 0.5
    y0f = jnp.floor(yy)
    x0f = jnp.floor(xx)
    wy = yy - y0f
    wx = xx - x0f
    y0 = jnp.clip(y0f, 0.0, float(FH - 1))
    y1 = jnp.minimum(y0 + 1.0, float(FH - 1))
    x0 = jnp.clip(x0f, 0.0, float(FW - 1))
    x1 = jnp.minimum(x0 + 1.0, float(FW - 1))
    lane = _fiota((1, NLOC), 1)
    B = jnp.zeros((BM, NLOC), jnp.float32)
    for (yi, xi, wgt) in ((y0, x0, (1 - wy) * (1 - wx)),
                          (y0, x1, (1 - wy) * wx),
                          (y1, x0, wy * (1 - wx)),
                          (y1, x1, wy * wx)):
        flat = yi * float(FW) + xi
        B += wgt * (lane == flat).astype(jnp.float32)
    o_ref[sl, :] = jnp.dot(B, feat_ref[...],
                           preferred_element_type=jnp.float32,
                           precision=jax.lax.Precision.HIGHEST)


def _roi_align_mm(props, featT):
    bxr = jnp.repeat(props, POOL * POOL, axis=0)
    M = POST_NMS * POOL * POOL
    bm = M // 8
    return pl.pallas_call(
        _roi_body,
        grid=(8,),
        in_specs=[
            pl.BlockSpec((bm, 4), lambda i: (i, 0)),
            pl.BlockSpec((NLOC, 2048), lambda i: (0, 0)),
        ],
        out_specs=pl.BlockSpec((bm, 2048), lambda i: (i, 0)),
        out_shape=jax.ShapeDtypeStruct((M, 2048), jnp.float32),
        compiler_params=pltpu.CompilerParams(
            dimension_semantics=("parallel",)),
    )(bxr, featT)


def _fc2h_body(x_ref, w2_ref, b2_ref, wh_ref, bh_ref, o_ref):
    h = jnp.maximum(jnp.dot(x_ref[...], w2_ref[...],
                            preferred_element_type=jnp.float32, precision=jax.lax.Precision.HIGHEST)
                    + b2_ref[...], 0.0)
    o_ref[...] = jnp.dot(h, wh_ref[...],
                         preferred_element_type=jnp.float32, precision=jax.lax.Precision.HIGHEST) + bh_ref[...]


def _fc2_heads(h1, fc2_w, fc2_b, wh, bh):
    NH = wh.shape[1]
    return pl.pallas_call(
        _fc2h_body,
        in_specs=[pl.BlockSpec(h1.shape, lambda: (0, 0)),
                  pl.BlockSpec(fc2_w.shape, lambda: (0, 0)),
                  pl.BlockSpec((1, 1024), lambda: (0, 0)),
                  pl.BlockSpec(wh.shape, lambda: (0, 0)),
                  pl.BlockSpec((1, NH), lambda: (0, 0))],
        out_specs=pl.BlockSpec((POST_NMS, NH), lambda: (0, 0)),
        out_shape=jax.ShapeDtypeStruct((POST_NMS, NH), jnp.float32),
    )(h1, fc2_w, fc2_b.reshape(1, 1024), wh, bh.reshape(1, NH))


def kernel(images, w_stem, w1, w2, w3, w4, w_rpn, b_rpn, w_obj, b_obj,
           w_reg, b_regc, fc1_w, fc1_b, fc2_w, fc2_b, cls_w, cls_b,
           box_w, box_b):
    f32 = jnp.float32
    img = jnp.transpose(images[0], (1, 2, 0))

    pat = _im2col(img, 7, 2, 3)
    pat = jnp.pad(pat, ((0, 0), (0, 13)))
    ws = jnp.pad(_wconv(w_stem), ((0, 13), (0, 0)))
    zeros64 = jnp.zeros((64,), f32)
    y = _matmul(pat, ws, zeros64, True, 4096, 64, 160)
    y = _maxpool(y, 256, 256, 64)

    pat = _im2col(y.reshape(128, 128, 64), 3, 1, 1)
    y = _matmul(pat, _wconv(w1), jnp.zeros((256,), f32), True, 2048, 256, 576)

    pat = _im2col(y.reshape(128, 128, 256), 3, 2, 1)
    y = _matmul(pat, _wconv(w2), jnp.zeros((512,), f32), True, 1024, 512, 2304)

    pat = _im2col(y.reshape(64, 64, 512), 3, 2, 1)
    y = _matmul(pat, _wconv(w3), jnp.zeros((1024,), f32), True, 1024, 1024, 1152)

    pat = _im2col(y.reshape(32, 32, 1024), 3, 1, 1)
    featT = _matmul(pat, _wconv(w4), jnp.zeros((2048,), f32), True,
                    1024, 2048, 1024)

    pat = _im2col(featT.reshape(32, 32, 2048), 3, 1, 1)
    t = _matmul(pat, _wconv(w_rpn), b_rpn, True, 1024, 2048, 1024)
    whead = jnp.concatenate([w_obj[:, :, 0, 0].T, w_reg[:, :, 0, 0].T], axis=1)
    whead = jnp.pad(whead, ((0, 0), (0, 4)))
    bhead = jnp.pad(jnp.concatenate([b_obj, b_regc]), (0, 4))
    ho = _matmul(t, whead, bhead, False, 1024, 64, 2048)
    scores = ho[:, :A].reshape(-1)
    deltas = ho[:, A:A + 4 * A].reshape(NANCH, 4)

    ranks = _rank(scores)
    dcat = jnp.concatenate([deltas, jnp.asarray(_ANCHORS)], axis=1)
    props = _nms_select(ranks, dcat)

    pooled = _roi_align_mm(props, featT)
    pooled = pooled.reshape(POST_NMS, POOL * POOL * 2048)

    fc1_wp = fc1_w.reshape(2048, POOL * POOL, 1024)
    fc1_wp = jnp.transpose(fc1_wp, (1, 0, 2)).reshape(POOL * POOL * 2048, 1024)
    h1 = _matmul(pooled, fc1_wp, fc1_b, True, POST_NMS, 1024, 2048)

    wh = jnp.concatenate([cls_w, box_w], axis=1)
    wh = jnp.pad(wh, ((0, 0), (0, 57)))
    bh = jnp.pad(jnp.concatenate([cls_b, box_b]), (0, 57))
    out = _fc2_heads(h1, fc2_w, fc2_b, wh, bh)
    return out[:, :5 * NUM_CLASSES]
